# speedup vs baseline: 3.2637x; 3.2637x over previous
import functools

import jax
import jax.numpy as jnp
from jax import lax
from jax.experimental import pallas as pl
from jax.experimental.pallas import tpu as pltpu
from jax.experimental.pallas import tpu_sc as plsc

N = 10000
D = 128
E = 320000
NW = 32
C = 128
CH = 80
EPT = CH * C
NP = 10240
RPT = NP // 16


def _sc_aggregate(h, src3, dst3, zeros1d, zeros2d, ones1):
    mesh = plsc.VectorSubcoreMesh(core_axis_name="c", subcore_axis_name="s")

    @functools.partial(
        pl.kernel,
        mesh=mesh,
        out_type=[
            jax.ShapeDtypeStruct((2, NP, D), jnp.float32),
            jax.ShapeDtypeStruct((2, NP), jnp.float32),
        ],
        scratch_types=[
            pltpu.VMEM((CH, C), jnp.int32),
            pltpu.VMEM((CH, C), jnp.int32),
            pltpu.VMEM((C, D), jnp.float32),
            pltpu.VMEM((C,), jnp.float32),
            pltpu.VMEM_SHARED((NP, D), jnp.float32),
            pltpu.VMEM_SHARED((NP,), jnp.float32),
            pltpu.SemaphoreType.DMA,
        ],
    )
    def k(h_hbm, src_hbm, dst_hbm, z1_hbm, z2_hbm, on_hbm,
          agg_out, deg_out,
          src_v, dst_v, rows_v, ones_v, acc_sh, deg_sh, sem):
        cid = lax.axis_index("c")
        sid = lax.axis_index("s")
        wid = cid * 16 + sid
        base = sid * RPT

        pltpu.sync_copy(z2_hbm, rows_v)
        for t in range(RPT // C):
            pltpu.sync_copy(rows_v, acc_sh.at[pl.ds(base + t * C, C)])
        pltpu.sync_copy(z1_hbm.at[pl.ds(base, RPT)], deg_sh.at[pl.ds(base, RPT)])
        pltpu.sync_copy(on_hbm, ones_v)
        pltpu.sync_copy(src_hbm.at[wid], src_v)
        pltpu.sync_copy(dst_hbm.at[wid], dst_v)
        plsc.subcore_barrier()

        def body(j, carry):
            pltpu.async_copy(h_hbm.at[src_v.at[j]], rows_v, sem).wait()
            pltpu.sync_copy(rows_v, acc_sh.at[dst_v.at[j]], add=True)
            pltpu.sync_copy(ones_v, deg_sh.at[dst_v.at[j]], add=True)
            return carry

        lax.fori_loop(0, CH, body, 0)
        plsc.subcore_barrier()

        pltpu.sync_copy(acc_sh.at[pl.ds(base, RPT)],
                        agg_out.at[cid, pl.ds(base, RPT)])
        pltpu.sync_copy(deg_sh.at[pl.ds(base, RPT)],
                        deg_out.at[cid, pl.ds(base, RPT)])

    return k(h, src3, dst3, zeros1d, zeros2d, ones1)


def _tc_combine(pp, dd, hx, Wl, Wr, b, relu):
    def body(pp_ref, dd_ref, x_ref, wl_ref, wr_ref, b_ref, o_ref):
        agg = pp_ref[0] + pp_ref[1]
        deg = dd_ref[0] + dd_ref[1]
        inv = 1.0 / jnp.maximum(deg, 1.0)
        mean = agg * inv[:, None]
        acc = jnp.dot(mean, wl_ref[...], preferred_element_type=jnp.float32)
        acc = acc + jnp.dot(x_ref[...], wr_ref[...],
                            preferred_element_type=jnp.float32)
        acc = acc + b_ref[...][None, :]
        if relu:
            acc = jnp.maximum(acc, 0.0)
        o_ref[...] = acc

    return pl.pallas_call(
        body,
        out_shape=jax.ShapeDtypeStruct((N, D), jnp.float32),
    )(pp, dd, hx, Wl, Wr, b)


def kernel(x, edge_index, W1l, W1r, b1, W2l, W2r, b2):
    src = edge_index[0]
    dst = edge_index[1]
    pad = NW * EPT - E
    src3 = jnp.concatenate([src, jnp.zeros((pad,), jnp.int32)]).reshape(NW, CH, C)
    dst3 = jnp.concatenate([dst, jnp.full((pad,), NP - 1, jnp.int32)]).reshape(NW, CH, C)
    zeros1d = jnp.zeros((NP,), jnp.float32)
    zeros2d = jnp.zeros((C, D), jnp.float32)
    ones1 = jnp.ones((C,), jnp.float32)

    pp1, dd = _sc_aggregate(x, src3, dst3, zeros1d, zeros2d, ones1)
    dds = dd[:, :N]
    h1 = _tc_combine(pp1[:, :N], dds, x, W1l, W1r, b1, True)
    pp2, _ = _sc_aggregate(h1, src3, dst3, zeros1d, zeros2d, ones1)
    out = _tc_combine(pp2[:, :N], dds, h1, W2l, W2r, b2, False)
    return out
